# baseline (speedup 1.0000x reference)
import functools

import jax
import jax.numpy as jnp
from jax import lax
from jax.experimental import pallas as pl
from jax.experimental.pallas import tpu as pltpu
from jax.experimental.pallas import tpu_sc as plsc

_NUM_CORES = 2
_NUM_SUBCORES = 16
_NUM_WORKERS = _NUM_CORES * _NUM_SUBCORES

_CHUNK = 512
_NBUF = 4

_STRIPE = 128


def _make_table_rowmajor(vocab: int, embed: int):
  blk = 4096
  pack = _STRIPE // embed

  def body(in_ref, out_ref):
    x = in_ref[...]
    ident = jnp.eye(embed, dtype=jnp.float32)
    y = lax.dot_general(
        x, ident, (((0,), (0,)), ((), ())),
        preferred_element_type=jnp.float32)
    y3 = y.reshape(blk // pack, pack, embed)
    out_ref[...] = jnp.concatenate(
        [y3[:, k, :] for k in range(pack)], axis=1)

  return pl.pallas_call(
      body,
      grid=(pl.cdiv(vocab, blk),),
      in_specs=[pl.BlockSpec((embed, blk), lambda q: (0, q))],
      out_specs=pl.BlockSpec((blk // pack, _STRIPE), lambda q: (q, 0)),
      out_shape=jax.ShapeDtypeStruct((vocab * embed // _STRIPE, _STRIPE),
                                     jnp.float32),
  )


def _make_gather(num_rows: int, vocab: int, embed: int):
  assert num_rows % (_NUM_WORKERS * _CHUNK) == 0
  rows_per_worker = num_rows // _NUM_WORKERS
  n_chunks = rows_per_worker // _CHUNK
  ahead = _NBUF - 1
  assert n_chunks >= ahead

  mesh = plsc.VectorSubcoreMesh(
      core_axis_name="c", subcore_axis_name="s")

  @functools.partial(
      pl.kernel,
      out_type=jax.ShapeDtypeStruct((num_rows, _STRIPE), jnp.float32),
      mesh=mesh,
      scratch_types=[
          pltpu.VMEM((n_chunks, _CHUNK), jnp.int32),
          pltpu.VMEM((_NBUF, _CHUNK, embed), jnp.float32),
          pltpu.SemaphoreType.DMA((_NBUF,)),
          pltpu.SemaphoreType.DMA((_NBUF,)),
      ],
      compiler_params=pltpu.CompilerParams(use_tc_tiling_on_sc=False),
  )
  def gather_kernel(idx_hbm, table_hbm, out_hbm, idx_v, rows_v, gsem, osem):
    wid = lax.axis_index("s") * _NUM_CORES + lax.axis_index("c")
    base = wid * rows_per_worker
    pltpu.sync_copy(idx_hbm.at[wid], idx_v)

    def gather_copy(j, b):
      return pltpu.make_async_copy(
          table_hbm.at[idx_v.at[j]], rows_v.at[b], gsem.at[b])

    def out_copy(j, b):
      return pltpu.make_async_copy(
          rows_v.at[b],
          out_hbm.at[pl.ds(base + j * _CHUNK, _CHUNK), pl.ds(0, embed)],
          osem.at[b])

    for j in range(ahead):
      gather_copy(j, j).start()

    def step(j, carry):
      b = lax.rem(j, _NBUF)
      bp = lax.rem(j + _NBUF - 1, _NBUF)
      gather_copy(j, b).wait()

      @pl.when(j >= 1)
      def _():
        out_copy(j - 1, bp).wait()

      @pl.when(j + ahead < n_chunks)
      def _():
        gather_copy(j + ahead, bp).start()

      out_copy(j, b).start()
      return carry

    lax.fori_loop(0, n_chunks, step, 0)

    out_copy(n_chunks - 1, (n_chunks - 1) % _NBUF).wait()

  return gather_kernel


def _make_relayout(fields: int, batch: int, embed: int):
  bt_blk = 32
  rows_blk = bt_blk * 128
  pack = _STRIPE // embed

  def body(in_ref, out_ref):
    x = in_ref[:, :embed]
    x = x.reshape(bt_blk, 128, embed)
    y = jnp.swapaxes(x, 1, 2)
    y = y.reshape(bt_blk, embed // 8, 8, 128)
    out_ref[0] = jnp.transpose(y, (1, 0, 2, 3))

  n_q = batch // rows_blk
  return pl.pallas_call(
      body,
      grid=(fields, n_q),
      in_specs=[
          pl.BlockSpec((rows_blk, _STRIPE),
                       lambda f, q: (f * n_q + q, 0))
      ],
      out_specs=pl.BlockSpec(
          (1, embed // 8, bt_blk, 8, 128), lambda f, q: (f, 0, q, 0, 0)),
      out_shape=jax.ShapeDtypeStruct(
          (fields, embed // 8, batch // 128, 8, 128), jnp.float32),
  )


def kernel(x, table):
  batch, fields = x.shape
  vocab, embed = table.shape
  num_rows = batch * fields
  idx = x.T.reshape(_NUM_WORKERS, num_rows // (_NUM_WORKERS * _CHUNK), _CHUNK)
  idx = idx.astype(jnp.int32)
  table_rm = _make_table_rowmajor(vocab, embed)(table.T)
  table_rm = table_rm.reshape(vocab, embed)
  flat = _make_gather(num_rows, vocab, embed)(idx, table_rm)
  out5 = _make_relayout(fields, batch, embed)(flat)
  out5 = out5.transpose(2, 4, 0, 1, 3)
  return out5.reshape(batch, fields, embed)


# speedup vs baseline: 1.0871x; 1.0871x over previous
import functools

import jax
import jax.numpy as jnp
from jax import lax
from jax.experimental import pallas as pl
from jax.experimental.pallas import tpu as pltpu
from jax.experimental.pallas import tpu_sc as plsc

_NUM_CORES = 2
_NUM_SUBCORES = 16
_NUM_WORKERS = _NUM_CORES * _NUM_SUBCORES

_CHUNK = 512
_NBUF = 4

_STRIPE = 128


def _make_table_rowmajor(vocab: int, embed: int):
  blk = 16384
  pack = _STRIPE // embed

  def body(in_ref, out_ref):
    y = in_ref[...].T
    y3 = y.reshape(blk // pack, pack, embed)
    out_ref[...] = jnp.concatenate(
        [y3[:, k, :] for k in range(pack)], axis=1)

  return pl.pallas_call(
      body,
      grid=(pl.cdiv(vocab, blk),),
      in_specs=[pl.BlockSpec((embed, blk), lambda q: (0, q))],
      out_specs=pl.BlockSpec((blk // pack, _STRIPE), lambda q: (q, 0)),
      out_shape=jax.ShapeDtypeStruct((vocab * embed // _STRIPE, _STRIPE),
                                     jnp.float32),
  )


def _make_gather(num_rows: int, vocab: int, embed: int):
  assert num_rows % (_NUM_WORKERS * _CHUNK) == 0
  rows_per_worker = num_rows // _NUM_WORKERS
  n_chunks = rows_per_worker // _CHUNK
  ahead = _NBUF - 1
  assert n_chunks >= ahead

  mesh = plsc.VectorSubcoreMesh(
      core_axis_name="c", subcore_axis_name="s")

  @functools.partial(
      pl.kernel,
      out_type=jax.ShapeDtypeStruct((num_rows, _STRIPE), jnp.float32),
      mesh=mesh,
      scratch_types=[
          pltpu.VMEM((n_chunks, _CHUNK), jnp.int32),
          pltpu.VMEM((_NBUF, _CHUNK, embed), jnp.float32),
          pltpu.SemaphoreType.DMA((_NBUF,)),
          pltpu.SemaphoreType.DMA((_NBUF,)),
      ],
      compiler_params=pltpu.CompilerParams(use_tc_tiling_on_sc=False),
  )
  def gather_kernel(idx_hbm, table_hbm, out_hbm, idx_v, rows_v, gsem, osem):
    wid = lax.axis_index("s") * _NUM_CORES + lax.axis_index("c")
    base = wid * rows_per_worker
    pltpu.sync_copy(idx_hbm.at[wid], idx_v)

    def gather_copy(j, b):
      return pltpu.make_async_copy(
          table_hbm.at[idx_v.at[j]], rows_v.at[b], gsem.at[b])

    def out_copy(j, b):
      return pltpu.make_async_copy(
          rows_v.at[b],
          out_hbm.at[pl.ds(base + j * _CHUNK, _CHUNK), pl.ds(0, embed)],
          osem.at[b])

    for j in range(ahead):
      gather_copy(j, j).start()

    def step(j, carry):
      b = lax.rem(j, _NBUF)
      bp = lax.rem(j + _NBUF - 1, _NBUF)
      gather_copy(j, b).wait()

      @pl.when(j >= 1)
      def _():
        out_copy(j - 1, bp).wait()

      @pl.when(j + ahead < n_chunks)
      def _():
        gather_copy(j + ahead, bp).start()

      out_copy(j, b).start()
      return carry

    lax.fori_loop(0, n_chunks, step, 0)

    out_copy(n_chunks - 1, (n_chunks - 1) % _NBUF).wait()

  return gather_kernel


def _make_relayout(fields: int, batch: int, embed: int):
  bt_blk = 32
  rows_blk = bt_blk * 128
  pack = _STRIPE // embed

  def body(in_ref, out_ref):
    x = in_ref[:, :embed]
    x = x.reshape(bt_blk, 128, embed)
    y = jnp.swapaxes(x, 1, 2)
    y = y.reshape(bt_blk, embed // 8, 8, 128)
    out_ref[0] = jnp.transpose(y, (1, 0, 2, 3))

  n_q = batch // rows_blk
  return pl.pallas_call(
      body,
      grid=(fields, n_q),
      in_specs=[
          pl.BlockSpec((rows_blk, _STRIPE),
                       lambda f, q: (f * n_q + q, 0))
      ],
      out_specs=pl.BlockSpec(
          (1, embed // 8, bt_blk, 8, 128), lambda f, q: (f, 0, q, 0, 0)),
      out_shape=jax.ShapeDtypeStruct(
          (fields, embed // 8, batch // 128, 8, 128), jnp.float32),
  )


def kernel(x, table):
  batch, fields = x.shape
  vocab, embed = table.shape
  num_rows = batch * fields
  idx = x.T.reshape(_NUM_WORKERS, num_rows // (_NUM_WORKERS * _CHUNK), _CHUNK)
  idx = idx.astype(jnp.int32)
  table_rm = _make_table_rowmajor(vocab, embed)(table.T)
  table_rm = table_rm.reshape(vocab, embed)
  flat = _make_gather(num_rows, vocab, embed)(idx, table_rm)
  out5 = _make_relayout(fields, batch, embed)(flat)
  out5 = out5.transpose(2, 4, 0, 1, 3)
  return out5.reshape(batch, fields, embed)


# speedup vs baseline: 1.9769x; 1.8186x over previous
import functools

import jax
import jax.numpy as jnp
from jax import lax
from jax.experimental import pallas as pl
from jax.experimental.pallas import tpu as pltpu
from jax.experimental.pallas import tpu_sc as plsc

_NUM_CORES = 2
_NUM_SUBCORES = 16
_NUM_WORKERS = _NUM_CORES * _NUM_SUBCORES

_CHUNK = 512
_NBUF = 4

_STRIPE = 128
_ABLK = 16384


def _make_table_rowmajor(vocab: int, embed: int):
  blk = _ABLK
  pack = _STRIPE // embed

  def body(in_ref, out_ref):
    g = in_ref[...].reshape(_STRIPE, blk // pack)
    y = g.T
    perm = jnp.arange(_STRIPE, dtype=jnp.int32)
    perm = ((perm % embed) * pack + perm // embed)
    out_ref[...] = jnp.take_along_axis(
        y, jnp.broadcast_to(perm[None, :], y.shape), axis=1)

  n_blocks = pl.cdiv(vocab, blk)
  return pl.pallas_call(
      body,
      grid=(n_blocks,),
      in_specs=[pl.BlockSpec((embed, blk), lambda q: (0, q))],
      out_specs=pl.BlockSpec((blk // pack, _STRIPE), lambda q: (q, 0)),
      out_shape=jax.ShapeDtypeStruct((n_blocks * blk // pack, _STRIPE),
                                     jnp.float32),
  )


def _make_gather(num_rows: int, vocab: int, embed: int):
  assert num_rows % (_NUM_WORKERS * _CHUNK) == 0
  rows_per_worker = num_rows // _NUM_WORKERS
  n_chunks = rows_per_worker // _CHUNK
  ahead = _NBUF - 1
  assert n_chunks >= ahead

  mesh = plsc.VectorSubcoreMesh(
      core_axis_name="c", subcore_axis_name="s")

  @functools.partial(
      pl.kernel,
      out_type=jax.ShapeDtypeStruct((num_rows, _STRIPE), jnp.float32),
      mesh=mesh,
      scratch_types=[
          pltpu.VMEM((n_chunks, _CHUNK), jnp.int32),
          pltpu.VMEM((_NBUF, _CHUNK, embed), jnp.float32),
          pltpu.SemaphoreType.DMA((_NBUF,)),
          pltpu.SemaphoreType.DMA((_NBUF,)),
      ],
      compiler_params=pltpu.CompilerParams(use_tc_tiling_on_sc=False),
  )
  def gather_kernel(idx_hbm, table_hbm, out_hbm, idx_v, rows_v, gsem, osem):
    wid = lax.axis_index("s") * _NUM_CORES + lax.axis_index("c")
    base = wid * rows_per_worker
    pltpu.sync_copy(idx_hbm.at[wid], idx_v)

    def gather_copy(j, b):
      return pltpu.make_async_copy(
          table_hbm.at[idx_v.at[j]], rows_v.at[b], gsem.at[b])

    def out_copy(j, b):
      return pltpu.make_async_copy(
          rows_v.at[b],
          out_hbm.at[pl.ds(base + j * _CHUNK, _CHUNK), pl.ds(0, embed)],
          osem.at[b])

    for j in range(ahead):
      gather_copy(j, j).start()

    def step(j, carry):
      b = lax.rem(j, _NBUF)
      bp = lax.rem(j + _NBUF - 1, _NBUF)
      gather_copy(j, b).wait()

      @pl.when(j >= 1)
      def _():
        out_copy(j - 1, bp).wait()

      @pl.when(j + ahead < n_chunks)
      def _():
        gather_copy(j + ahead, bp).start()

      out_copy(j, b).start()
      return carry

    lax.fori_loop(0, n_chunks, step, 0)

    out_copy(n_chunks - 1, (n_chunks - 1) % _NBUF).wait()

  return gather_kernel


def _make_relayout(fields: int, batch: int, embed: int):
  bt_blk = 32
  rows_blk = bt_blk * 128
  pack = _STRIPE // embed

  def body(in_ref, out_ref):
    x = in_ref[:, :embed]
    x = x.reshape(bt_blk, 128, embed)
    y = jnp.swapaxes(x, 1, 2)
    y = y.reshape(bt_blk, embed // 8, 8, 128)
    out_ref[0] = jnp.transpose(y, (1, 0, 2, 3))

  n_q = batch // rows_blk
  return pl.pallas_call(
      body,
      grid=(fields, n_q),
      in_specs=[
          pl.BlockSpec((rows_blk, _STRIPE),
                       lambda f, q: (f * n_q + q, 0))
      ],
      out_specs=pl.BlockSpec(
          (1, embed // 8, bt_blk, 8, 128), lambda f, q: (f, 0, q, 0, 0)),
      out_shape=jax.ShapeDtypeStruct(
          (fields, embed // 8, batch // 128, 8, 128), jnp.float32),
  )


def kernel(x, table):
  batch, fields = x.shape
  vocab, embed = table.shape
  num_rows = batch * fields
  pack = _STRIPE // embed
  b4 = _ABLK // pack
  idx = x.T.astype(jnp.int32)
  q, r = jnp.divmod(idx, _ABLK)
  k, j = jnp.divmod(r, b4)
  idx = (q * b4 + j) * pack + k
  idx = idx.reshape(_NUM_WORKERS, num_rows // (_NUM_WORKERS * _CHUNK), _CHUNK)
  table_rm = _make_table_rowmajor(vocab, embed)(table.T)
  table_rm = table_rm.reshape(-1, embed)
  flat = _make_gather(num_rows, table_rm.shape[0], embed)(idx, table_rm)
  out5 = _make_relayout(fields, batch, embed)(flat)
  out5 = out5.transpose(2, 4, 0, 1, 3)
  return out5.reshape(batch, fields, embed)


# speedup vs baseline: 1.9807x; 1.0019x over previous
import functools

import jax
import jax.numpy as jnp
from jax import lax
from jax.experimental import pallas as pl
from jax.experimental.pallas import tpu as pltpu
from jax.experimental.pallas import tpu_sc as plsc

_NUM_CORES = 2
_NUM_SUBCORES = 16
_NUM_WORKERS = _NUM_CORES * _NUM_SUBCORES

_CHUNK = 512
_NBUF = 4

_STRIPE = 128
_ABLK = 16384


def _make_table_rowmajor(vocab: int, embed: int):
  blk = _ABLK
  pack = _STRIPE // embed

  def body(in_ref, out_ref):
    g = in_ref[...].reshape(_STRIPE, blk // pack)
    y = g.T
    perm = jnp.arange(_STRIPE, dtype=jnp.int32)
    perm = ((perm % embed) * pack + perm // embed)
    out_ref[...] = jnp.take_along_axis(
        y, jnp.broadcast_to(perm[None, :], y.shape), axis=1)

  n_blocks = pl.cdiv(vocab, blk)
  return pl.pallas_call(
      body,
      grid=(n_blocks,),
      in_specs=[pl.BlockSpec((embed, blk), lambda q: (0, q))],
      out_specs=pl.BlockSpec((blk // pack, _STRIPE), lambda q: (q, 0)),
      out_shape=jax.ShapeDtypeStruct((n_blocks * blk // pack, _STRIPE),
                                     jnp.float32),
  )


def _make_gather(num_rows: int, vocab: int, embed: int):
  assert num_rows % (_NUM_WORKERS * _CHUNK) == 0
  rows_per_worker = num_rows // _NUM_WORKERS
  n_chunks = rows_per_worker // _CHUNK
  ahead = _NBUF - 1
  assert n_chunks >= ahead

  mesh = plsc.VectorSubcoreMesh(
      core_axis_name="c", subcore_axis_name="s")

  @functools.partial(
      pl.kernel,
      out_type=jax.ShapeDtypeStruct((num_rows, _STRIPE), jnp.float32),
      mesh=mesh,
      scratch_types=[
          pltpu.VMEM((n_chunks, _CHUNK), jnp.int32),
          pltpu.VMEM((_NBUF, _CHUNK, embed), jnp.float32),
          pltpu.SemaphoreType.DMA((_NBUF,)),
          pltpu.SemaphoreType.DMA((_NBUF,)),
      ],
      compiler_params=pltpu.CompilerParams(use_tc_tiling_on_sc=False),
  )
  def gather_kernel(idx_hbm, table_hbm, out_hbm, idx_v, rows_v, gsem, osem):
    wid = lax.axis_index("s") * _NUM_CORES + lax.axis_index("c")
    base = wid * rows_per_worker
    pltpu.sync_copy(idx_hbm.at[wid], idx_v)

    def gather_copy(j, b):
      return pltpu.make_async_copy(
          table_hbm.at[idx_v.at[j]], rows_v.at[b], gsem.at[b])

    def out_copy(j, b):
      return pltpu.make_async_copy(
          rows_v.at[b],
          out_hbm.at[pl.ds(base + j * _CHUNK, _CHUNK), pl.ds(0, embed)],
          osem.at[b])

    for j in range(ahead):
      gather_copy(j, j).start()

    def step(j, carry):
      b = lax.rem(j, _NBUF)
      bp = lax.rem(j + _NBUF - 1, _NBUF)
      gather_copy(j, b).wait()

      @pl.when(j >= 1)
      def _():
        out_copy(j - 1, bp).wait()

      @pl.when(j + ahead < n_chunks)
      def _():
        gather_copy(j + ahead, bp).start()

      out_copy(j, b).start()
      return carry

    lax.fori_loop(0, n_chunks, step, 0)

    out_copy(n_chunks - 1, (n_chunks - 1) % _NBUF).wait()

  return gather_kernel


def _make_relayout(fields: int, batch: int, embed: int):
  bt_blk = 32
  rows_blk = bt_blk * 128
  pack = _STRIPE // embed

  def body(in_ref, out_ref):
    x = in_ref[...]
    y = x.T
    y = y.reshape(_STRIPE, bt_blk, 128)[:embed]
    y = y.reshape(embed // 8, 8, bt_blk, 128)
    out_ref[0] = jnp.transpose(y, (0, 2, 1, 3))

  n_q = batch // rows_blk
  return pl.pallas_call(
      body,
      grid=(fields, n_q),
      in_specs=[
          pl.BlockSpec((rows_blk, _STRIPE),
                       lambda f, q: (f * n_q + q, 0))
      ],
      out_specs=pl.BlockSpec(
          (1, embed // 8, bt_blk, 8, 128), lambda f, q: (f, 0, q, 0, 0)),
      out_shape=jax.ShapeDtypeStruct(
          (fields, embed // 8, batch // 128, 8, 128), jnp.float32),
  )


def kernel(x, table):
  batch, fields = x.shape
  vocab, embed = table.shape
  num_rows = batch * fields
  pack = _STRIPE // embed
  b4 = _ABLK // pack
  idx = x.T.astype(jnp.int32)
  q, r = jnp.divmod(idx, _ABLK)
  k, j = jnp.divmod(r, b4)
  idx = (q * b4 + j) * pack + k
  idx = idx.reshape(_NUM_WORKERS, num_rows // (_NUM_WORKERS * _CHUNK), _CHUNK)
  table_rm = _make_table_rowmajor(vocab, embed)(table.T)
  table_rm = table_rm.reshape(-1, embed)
  flat = _make_gather(num_rows, table_rm.shape[0], embed)(idx, table_rm)
  out5 = _make_relayout(fields, batch, embed)(flat)
  out5 = out5.transpose(2, 4, 0, 1, 3)
  return out5.reshape(batch, fields, embed)
